# baseline (speedup 1.0000x reference)
import functools
import numpy as np
import jax
import jax.numpy as jnp
from jax import lax
from jax.experimental import pallas as pl
from jax.experimental.pallas import tpu as pltpu
from jax.experimental.pallas import tpu_sc as plsc

VOCAB = 64
EMBED = 512
SEQ = 512
BATCH = 128
SCALE = float(np.sqrt(EMBED))

NW = 32
B_PER_W = BATCH // NW
S_CH = 32
NSC = SEQ // S_CH
NIT = NSC * B_PER_W
NBUF = 4
L = 16


def _pos_encoding(length, depth):
    half = depth // 2
    positions = np.arange(length)[:, np.newaxis]
    depths = np.arange(half)[np.newaxis, :] / half
    angle_rates = 1 / 10000 ** depths
    angle_rads = positions * angle_rates
    return np.concatenate(
        [np.sin(angle_rads), np.cos(angle_rads)], axis=-1
    ).astype(np.float32)


def _scale_body(table_ref, out_ref):
    out_ref[...] = table_ref[...] * SCALE


def _scaled_table(table):
    return pl.pallas_call(
        _scale_body,
        out_shape=jax.ShapeDtypeStruct((VOCAB, EMBED), jnp.float32),
    )(table)


def _sc_body(
    x_hbm, table_hbm, pos_hbm, out_hbm,
    pos_v, idx_v, rows_v, gsem, ssem, psem,
):
    wid = lax.axis_index("s") * 2 + lax.axis_index("c")
    b0 = wid * B_PER_W

    pltpu.sync_copy(x_hbm.at[pl.ds(b0 * SEQ, B_PER_W * SEQ)], idx_v)

    def pos_load(sc):
        return pltpu.make_async_copy(
            pos_hbm.at[pl.ds(sc * S_CH, S_CH), :],
            pos_v.at[lax.rem(sc, 2)],
            psem.at[lax.rem(sc, 2)],
        )

    def gather(c):
        sc = c // B_PER_W
        j = lax.rem(c, B_PER_W)
        buf = lax.rem(c, NBUF)
        return pltpu.make_async_copy(
            table_hbm.at[idx_v.at[pl.ds(j * SEQ + sc * S_CH, S_CH)]],
            rows_v.at[buf],
            gsem.at[buf],
        )

    def scatter(c):
        sc = c // B_PER_W
        j = lax.rem(c, B_PER_W)
        buf = lax.rem(c, NBUF)
        return pltpu.make_async_copy(
            rows_v.at[buf],
            out_hbm.at[b0 + j, pl.ds(sc * S_CH, S_CH), :],
            ssem.at[buf],
        )

    pos_load(0).start()
    gather(0).start()

    def chunk(c, _):
        sc = c // B_PER_W
        buf = lax.rem(c, NBUF)

        @pl.when(lax.rem(c, B_PER_W) == 0)
        def _():
            pos_load(sc).wait()

            @pl.when(sc + 1 < NSC)
            def _():
                pos_load(sc + 1).start()

        @pl.when(c + 1 < NIT)
        def _():
            @pl.when(c + 1 >= NBUF)
            def _():
                scatter(c + 1 - NBUF).wait()

            gather(c + 1).start()

        gather(c).wait()

        pbuf = lax.rem(sc, 2)

        def add_row(t):
            for k in range(EMBED // L):
                sl = pl.ds(k * L, L)
                p = pos_v[pbuf, t, sl]
                plsc.addupdate(rows_v.at[buf, t, sl], p)

        plsc.parallel_loop(0, S_CH, 1, unroll=4)(add_row)

        scatter(c).start()
        return 0

    lax.fori_loop(0, NIT, chunk, 0, unroll=False)

    def drain(c, _):
        scatter(c).wait()
        return 0

    lax.fori_loop(NIT - NBUF, NIT, drain, 0, unroll=False)


def kernel(x, table):
    pos = jnp.asarray(_pos_encoding(SEQ, EMBED))
    tbl = _scaled_table(table)
    xf = x.astype(jnp.int32).reshape(-1)

    mesh = plsc.VectorSubcoreMesh(core_axis_name="c", subcore_axis_name="s")
    sc = pl.kernel(
        _sc_body,
        mesh=mesh,
        out_type=jax.ShapeDtypeStruct((BATCH, SEQ, EMBED), jnp.float32),
        scratch_types=[
            pltpu.VMEM((2, S_CH, EMBED), jnp.float32),
            pltpu.VMEM((B_PER_W * SEQ,), jnp.int32),
            pltpu.VMEM((NBUF, S_CH, EMBED), jnp.float32),
            pltpu.SemaphoreType.DMA((NBUF,)),
            pltpu.SemaphoreType.DMA((NBUF,)),
            pltpu.SemaphoreType.DMA((2,)),
        ],
    )
    return sc(xf, tbl, pos)


# speedup vs baseline: 1.0789x; 1.0789x over previous
import functools
import numpy as np
import jax
import jax.numpy as jnp
from jax import lax
from jax.experimental import pallas as pl
from jax.experimental.pallas import tpu as pltpu
from jax.experimental.pallas import tpu_sc as plsc

VOCAB = 64
EMBED = 512
SEQ = 512
BATCH = 128
SCALE = float(np.sqrt(EMBED))

NW = 32
B_PER_W = BATCH // NW
S_CH = 32
NSC = SEQ // S_CH
NIT = NSC * B_PER_W
NBUF = 2
L = 16


def _pos_encoding(length, depth):
    half = depth // 2
    positions = np.arange(length)[:, np.newaxis]
    depths = np.arange(half)[np.newaxis, :] / half
    angle_rates = 1 / 10000 ** depths
    angle_rads = positions * angle_rates
    return np.concatenate(
        [np.sin(angle_rads), np.cos(angle_rads)], axis=-1
    ).astype(np.float32)


def _scale_body(table_ref, out_ref):
    out_ref[...] = table_ref[...] * SCALE


def _scaled_table(table):
    return pl.pallas_call(
        _scale_body,
        out_shape=jax.ShapeDtypeStruct((VOCAB, EMBED), jnp.float32),
    )(table)


def _sc_body(
    x_hbm, table_hbm, pos_hbm, out_hbm,
    pos_v, idx_v, rows_v, rows_sh, gsem, ssem, psem,
):
    sid = lax.axis_index("s")
    wid = sid * 2 + lax.axis_index("c")
    b0 = wid * B_PER_W

    pltpu.sync_copy(x_hbm.at[pl.ds(b0 * SEQ, B_PER_W * SEQ)], idx_v)

    def pos_load(sc):
        return pltpu.make_async_copy(
            pos_hbm.at[pl.ds(sc * S_CH, S_CH), :],
            pos_v.at[lax.rem(sc, 2)],
            psem.at[lax.rem(sc, 2)],
        )

    def gather(c):
        sc = c // B_PER_W
        j = lax.rem(c, B_PER_W)
        buf = lax.rem(c, NBUF)
        return pltpu.make_async_copy(
            table_hbm.at[idx_v.at[pl.ds(j * SEQ + sc * S_CH, S_CH)]],
            rows_v.at[buf],
            gsem.at[buf],
        )

    def scatter(c):
        sc = c // B_PER_W
        j = lax.rem(c, B_PER_W)
        sbuf = lax.rem(c, 2)
        return pltpu.make_async_copy(
            rows_sh.at[sid, sbuf],
            out_hbm.at[b0 + j, pl.ds(sc * S_CH, S_CH), :],
            ssem.at[sbuf],
        )

    pos_load(0).start()
    gather(0).start()

    def chunk(c, _):
        sc = c // B_PER_W
        buf = lax.rem(c, NBUF)

        @pl.when(lax.rem(c, B_PER_W) == 0)
        def _():
            pos_load(sc).wait()

            @pl.when(sc + 1 < NSC)
            def _():
                pos_load(sc + 1).start()

        @pl.when(c + 1 < NIT)
        def _():
            gather(c + 1).start()

        gather(c).wait()

        pbuf = lax.rem(sc, 2)

        def add_row(t):
            for k in range(EMBED // L):
                sl = pl.ds(k * L, L)
                p = pos_v[pbuf, t, sl]
                plsc.addupdate(rows_v.at[buf, t, sl], p)

        plsc.parallel_loop(0, S_CH, 1, unroll=4)(add_row)

        @pl.when(c >= 2)
        def _():
            scatter(c - 2).wait()

        pltpu.sync_copy(rows_v.at[buf], rows_sh.at[sid, lax.rem(c, 2)])
        scatter(c).start()
        return 0

    lax.fori_loop(0, NIT, chunk, 0, unroll=False)

    def drain(c, _):
        scatter(c).wait()
        return 0

    lax.fori_loop(NIT - 2, NIT, drain, 0, unroll=False)


def kernel(x, table):
    pos = jnp.asarray(_pos_encoding(SEQ, EMBED))
    tbl = _scaled_table(table)
    xf = x.astype(jnp.int32).reshape(-1)

    mesh = plsc.VectorSubcoreMesh(core_axis_name="c", subcore_axis_name="s")
    sc = pl.kernel(
        _sc_body,
        mesh=mesh,
        out_type=jax.ShapeDtypeStruct((BATCH, SEQ, EMBED), jnp.float32),
        scratch_types=[
            pltpu.VMEM((2, S_CH, EMBED), jnp.float32),
            pltpu.VMEM((B_PER_W * SEQ,), jnp.int32),
            pltpu.VMEM((NBUF, S_CH, EMBED), jnp.float32),
            pltpu.VMEM_SHARED((16, 2, S_CH, EMBED), jnp.float32),
            pltpu.SemaphoreType.DMA((NBUF,)),
            pltpu.SemaphoreType.DMA((2,)),
            pltpu.SemaphoreType.DMA((2,)),
        ],
    )
    return sc(xf, tbl, pos)


# speedup vs baseline: 1.0838x; 1.0046x over previous
import functools
import numpy as np
import jax
import jax.numpy as jnp
from jax import lax
from jax.experimental import pallas as pl
from jax.experimental.pallas import tpu as pltpu
from jax.experimental.pallas import tpu_sc as plsc

VOCAB = 64
EMBED = 512
SEQ = 512
BATCH = 128
SCALE = float(np.sqrt(EMBED))

NW = 32
B_PER_W = BATCH // NW
S_CH = 32
NSC = SEQ // S_CH
NIT = NSC * B_PER_W
NBUF = 2
L = 16


def _pos_encoding(length, depth):
    half = depth // 2
    positions = np.arange(length)[:, np.newaxis]
    depths = np.arange(half)[np.newaxis, :] / half
    angle_rates = 1 / 10000 ** depths
    angle_rads = positions * angle_rates
    return np.concatenate(
        [np.sin(angle_rads), np.cos(angle_rads)], axis=-1
    ).astype(np.float32)


def _scale_body(table_ref, out_ref):
    out_ref[...] = table_ref[...] * SCALE


def _scaled_table(table):
    return pl.pallas_call(
        _scale_body,
        out_shape=jax.ShapeDtypeStruct((VOCAB, EMBED), jnp.float32),
    )(table)


def _sc_body(
    x_hbm, table_hbm, pos_hbm, out_hbm,
    pos_v, idx_v, rows_v, rows_sh, gsem, ssem, psem, csem,
):
    sid = lax.axis_index("s")
    wid = sid * 2 + lax.axis_index("c")
    b0 = wid * B_PER_W

    pltpu.sync_copy(x_hbm.at[pl.ds(b0 * SEQ, B_PER_W * SEQ)], idx_v)

    def pos_load(sc):
        return pltpu.make_async_copy(
            pos_hbm.at[pl.ds(sc * S_CH, S_CH), :],
            pos_v.at[lax.rem(sc, 2)],
            psem.at[lax.rem(sc, 2)],
        )

    def gather(c):
        sc = c // B_PER_W
        j = lax.rem(c, B_PER_W)
        buf = lax.rem(c, NBUF)
        return pltpu.make_async_copy(
            table_hbm.at[idx_v.at[pl.ds(j * SEQ + sc * S_CH, S_CH)]],
            rows_v.at[buf],
            gsem.at[buf],
        )

    def scatter(c):
        sc = c // B_PER_W
        j = lax.rem(c, B_PER_W)
        sbuf = lax.rem(c, 2)
        return pltpu.make_async_copy(
            rows_sh.at[sid, sbuf],
            out_hbm.at[b0 + j, pl.ds(sc * S_CH, S_CH), :],
            ssem.at[sbuf],
        )

    def stage(c):
        return pltpu.make_async_copy(
            rows_v.at[lax.rem(c, NBUF)],
            rows_sh.at[sid, lax.rem(c, 2)],
            csem.at[lax.rem(c, 2)],
        )

    pos_load(0).start()
    gather(0).start()

    def chunk(c, _):
        sc = c // B_PER_W
        buf = lax.rem(c, NBUF)

        @pl.when(c >= 1)
        def _():
            stage(c - 1).wait()
            scatter(c - 1).start()

        @pl.when(lax.rem(c, B_PER_W) == 0)
        def _():
            pos_load(sc).wait()

            @pl.when(sc + 1 < NSC)
            def _():
                pos_load(sc + 1).start()

        @pl.when(c + 1 < NIT)
        def _():
            gather(c + 1).start()

        gather(c).wait()

        pbuf = lax.rem(sc, 2)

        def add_row(t):
            for k in range(EMBED // L):
                sl = pl.ds(k * L, L)
                p = pos_v[pbuf, t, sl]
                plsc.addupdate(rows_v.at[buf, t, sl], p)

        plsc.parallel_loop(0, S_CH, 1, unroll=4)(add_row)

        @pl.when(c >= 2)
        def _():
            scatter(c - 2).wait()

        stage(c).start()
        return 0

    lax.fori_loop(0, NIT, chunk, 0, unroll=False)

    stage(NIT - 1).wait()
    scatter(NIT - 1).start()
    scatter(NIT - 2).wait()
    scatter(NIT - 1).wait()


def kernel(x, table):
    pos = jnp.asarray(_pos_encoding(SEQ, EMBED))
    tbl = _scaled_table(table)
    xf = x.astype(jnp.int32).reshape(-1)

    mesh = plsc.VectorSubcoreMesh(core_axis_name="c", subcore_axis_name="s")
    sc = pl.kernel(
        _sc_body,
        mesh=mesh,
        out_type=jax.ShapeDtypeStruct((BATCH, SEQ, EMBED), jnp.float32),
        scratch_types=[
            pltpu.VMEM((2, S_CH, EMBED), jnp.float32),
            pltpu.VMEM((B_PER_W * SEQ,), jnp.int32),
            pltpu.VMEM((NBUF, S_CH, EMBED), jnp.float32),
            pltpu.VMEM_SHARED((16, 2, S_CH, EMBED), jnp.float32),
            pltpu.SemaphoreType.DMA((NBUF,)),
            pltpu.SemaphoreType.DMA((2,)),
            pltpu.SemaphoreType.DMA((2,)),
            pltpu.SemaphoreType.DMA((2,)),
        ],
    )
    return sc(xf, tbl, pos)
